# Initial kernel scaffold; baseline (speedup 1.0000x reference)
import jax
import jax.numpy as jnp
from jax.experimental import pallas as pl


def kernel(s_e, adjacency_matrix, W1, b1, W2, b2):
    raise NotImplementedError("write your pallas kernel here")


# speedup vs baseline: 2.6934x; 2.6934x over previous
import jax
import jax.numpy as jnp
from jax.experimental import pallas as pl


def _subgraph_kernel(x_ref, adj_ref, w1_ref, b1_ref, w2t_ref, b2_ref,
                     s_ref, id_ref, flag_ref):
    N = 128
    w1 = w1_ref[:]
    b1 = b1_ref[:]
    w2t = w2t_ref[:]
    b2 = b2_ref[:]
    col = jax.lax.broadcasted_iota(jnp.int32, (1, N), 1)

    ids = []
    for row in range(2):
        x = x_ref[0, row]
        h = jnp.maximum(
            jax.lax.dot_general(x, w1, (((1,), (0,)), ((), ())),
                                preferred_element_type=jnp.float32) + b1, 0.0)
        s = jax.lax.dot_general(w2t, h, (((1,), (1,)), ((), ())),
                                preferred_element_type=jnp.float32) + b2
        adj = adj_ref[0, row:row + 1, :]
        kill_col = 1 - row
        adj = adj * (col != kill_col).astype(jnp.float32)
        sm = s * adj
        s_ref[0, row:row + 1, :] = sm
        mx = jnp.max(sm, axis=1, keepdims=True)
        cand = jnp.where(sm == mx, col, N)
        idx = jnp.min(cand, axis=1, keepdims=True)
        id_ref[0, row:row + 1, :] = idx
        ids.append(idx)

    a = ids[0] > 0
    b = ids[1] > 0
    flag_ref[0, :, :] = jnp.where(
        a & b, 3.0, jnp.where(a, 1.0, jnp.where(b, 2.0, 0.0))
    ).astype(jnp.float32)


def kernel(s_e, adjacency_matrix, W1, b1, W2, b2):
    B, N, _, D = s_e.shape
    b1v = b1.reshape(1, D)
    w2t = W2.reshape(D, 1).T
    b2v = jnp.broadcast_to(b2.reshape(1, 1), (1, N))
    adj2 = adjacency_matrix[:, :2]

    grid = (B,)
    out_shapes = (
        jax.ShapeDtypeStruct((B, 2, N), jnp.float32),
        jax.ShapeDtypeStruct((B, 2, 1), jnp.int32),
        jax.ShapeDtypeStruct((B, 1, 1), jnp.float32),
    )
    in_specs = [
        pl.BlockSpec((1, 2, N, D), lambda i: (i, 0, 0, 0)),
        pl.BlockSpec((1, 2, N), lambda i: (i, 0, 0)),
        pl.BlockSpec((D, D), lambda i: (0, 0)),
        pl.BlockSpec((1, D), lambda i: (0, 0)),
        pl.BlockSpec((1, D), lambda i: (0, 0)),
        pl.BlockSpec((1, N), lambda i: (0, 0)),
    ]
    out_specs = (
        pl.BlockSpec((1, 2, N), lambda i: (i, 0, 0)),
        pl.BlockSpec((1, 2, 1), lambda i: (i, 0, 0)),
        pl.BlockSpec((1, 1, 1), lambda i: (i, 0, 0)),
    )
    scores, ids, flag = pl.pallas_call(
        _subgraph_kernel,
        grid=grid,
        in_specs=in_specs,
        out_specs=out_specs,
        out_shape=out_shapes,
    )(s_e, adj2, W1, b1v, w2t, b2v)

    final_id = ids.reshape(B, 2)
    return final_id, scores, flag.reshape(B)


# speedup vs baseline: 3.6090x; 1.3400x over previous
import jax
import jax.numpy as jnp
from jax.experimental import pallas as pl


def _subgraph_kernel(x_ref, adj_ref, w1_ref, b1_ref, w2t_ref, b2_ref,
                     s_ref, id_ref, flag_ref):
    B, N, D = 8, 128, 128
    x = x_ref[:].reshape(B * 2 * N, D)
    h = jnp.maximum(
        jax.lax.dot_general(x, w1_ref[:], (((1,), (0,)), ((), ())),
                            preferred_element_type=jnp.float32) + b1_ref[:],
        0.0)
    s_all = jax.lax.dot_general(w2t_ref[:], h, (((1,), (1,)), ((), ())),
                                preferred_element_type=jnp.float32)
    lane = jax.lax.broadcasted_iota(jnp.int32, (1, 2 * N), 1)
    j = jax.lax.rem(lane, N)
    row = jax.lax.rem(lane // N, 2)
    killed = ((row == 0) & (j == 1)) | ((row == 1) & (j == 0))
    kill_mask = jnp.where(killed, 0.0, 1.0)

    col = jax.lax.broadcasted_iota(jnp.int32, (1, N), 1)
    sub_ids, obj_ids = [], []
    for b in range(B):
        seg = s_all[:, b * 2 * N:(b + 1) * 2 * N] + b2_ref[:]
        adj = adj_ref[b:b + 1, :] * kill_mask
        sm = seg * adj
        s_ref[b:b + 1, :] = sm
        for r, acc in ((0, sub_ids), (1, obj_ids)):
            smr = sm[:, r * N:(r + 1) * N]
            mx = jnp.max(smr, axis=1, keepdims=True)
            cand = jnp.where(smr == mx, col, N)
            acc.append(jnp.min(cand, axis=1, keepdims=True))
    sub = jnp.concatenate(sub_ids, axis=1)
    obj = jnp.concatenate(obj_ids, axis=1)
    id_ref[:] = jnp.concatenate([sub, obj], axis=1)
    a = sub > 0
    bb = obj > 0
    flag_ref[:] = jnp.where(a & bb, 3.0,
                            jnp.where(a, 1.0,
                                      jnp.where(bb, 2.0, 0.0))
                            ).astype(jnp.float32)


def kernel(s_e, adjacency_matrix, W1, b1, W2, b2):
    B, N, _, D = s_e.shape
    b1v = b1.reshape(1, D)
    w2t = W2.reshape(D, 1).T
    b2v = jnp.broadcast_to(b2.reshape(1, 1), (1, 2 * N))
    adj2 = adjacency_matrix[:, :2].reshape(B, 2 * N)

    out_shapes = (
        jax.ShapeDtypeStruct((B, 2 * N), jnp.float32),
        jax.ShapeDtypeStruct((1, 2 * B), jnp.int32),
        jax.ShapeDtypeStruct((1, B), jnp.float32),
    )
    in_specs = [
        pl.BlockSpec((B, 2, N, D), lambda i: (0, 0, 0, 0)),
        pl.BlockSpec((B, 2 * N), lambda i: (0, 0)),
        pl.BlockSpec((D, D), lambda i: (0, 0)),
        pl.BlockSpec((1, D), lambda i: (0, 0)),
        pl.BlockSpec((1, D), lambda i: (0, 0)),
        pl.BlockSpec((1, 2 * N), lambda i: (0, 0)),
    ]
    out_specs = (
        pl.BlockSpec((B, 2 * N), lambda i: (0, 0)),
        pl.BlockSpec((1, 2 * B), lambda i: (0, 0)),
        pl.BlockSpec((1, B), lambda i: (0, 0)),
    )
    scores, ids, flag = pl.pallas_call(
        _subgraph_kernel,
        grid=(1,),
        in_specs=in_specs,
        out_specs=out_specs,
        out_shape=out_shapes,
    )(s_e, adj2, W1, b1v, w2t, b2v)

    final_id = ids.reshape(2, B).T
    return final_id, scores.reshape(B, 2, N), flag.reshape(B)


# speedup vs baseline: 3.7543x; 1.0402x over previous
import jax
import jax.numpy as jnp
from jax.experimental import pallas as pl


def _subgraph_kernel(x_ref, adj_ref, w1_ref, b1_ref, w2_ref, b2_ref,
                     s_ref, id_ref, flag_ref):
    B, N, D = 8, 128, 128
    x = x_ref[:].reshape(B * 2 * N, D)
    h = jnp.maximum(
        jax.lax.dot_general(x, w1_ref[:], (((1,), (0,)), ((), ())),
                            preferred_element_type=jnp.float32) + b1_ref[:],
        0.0)
    s_all = jax.lax.dot_general(w2_ref[:], h, (((0,), (1,)), ((), ())),
                                preferred_element_type=jnp.float32)
    lane = jax.lax.broadcasted_iota(jnp.int32, (1, 2 * N), 1)
    j = jax.lax.rem(lane, N)
    row = lane // N
    killed = ((row == 0) & (j == 1)) | ((row == 1) & (j == 0))
    kill_mask = jnp.where(killed, 0.0, 1.0)

    col = jax.lax.broadcasted_iota(jnp.int32, (1, N), 1)
    for b in range(B):
        seg = s_all[:, b * 2 * N:(b + 1) * 2 * N] + b2_ref[:]
        adj = jnp.concatenate(
            [adj_ref[b, 0:1, :], adj_ref[b, 1:2, :]], axis=1) * kill_mask
        sm = seg * adj
        s_ref[b:b + 1, :] = sm
        ids = []
        for r in range(2):
            smr = sm[:, r * N:(r + 1) * N]
            mx = jnp.max(smr, axis=1, keepdims=True)
            cand = jnp.where(smr == mx, col, N)
            idx = jnp.min(cand, axis=1, keepdims=True)
            id_ref[b:b + 1, r:r + 1] = idx
            ids.append(idx)
        a = ids[0] > 0
        bb = ids[1] > 0
        flag_ref[b:b + 1, :] = jnp.where(
            a & bb, 3.0, jnp.where(a, 1.0, jnp.where(bb, 2.0, 0.0))
        ).astype(jnp.float32)


def kernel(s_e, adjacency_matrix, W1, b1, W2, b2):
    B, N, _, D = s_e.shape
    out_shapes = (
        jax.ShapeDtypeStruct((B, 2 * N), jnp.float32),
        jax.ShapeDtypeStruct((B, 2), jnp.int32),
        jax.ShapeDtypeStruct((B, 1), jnp.float32),
    )
    in_specs = [
        pl.BlockSpec((B, 2, N, D), lambda i: (0, 0, 0, 0)),
        pl.BlockSpec((B, 8, N), lambda i: (0, 0, 0)),
        pl.BlockSpec((D, D), lambda i: (0, 0)),
        pl.BlockSpec((1, D), lambda i: (0, 0)),
        pl.BlockSpec((D, 1), lambda i: (0, 0)),
        pl.BlockSpec((1, 1), lambda i: (0, 0)),
    ]
    out_specs = (
        pl.BlockSpec((B, 2 * N), lambda i: (0, 0)),
        pl.BlockSpec((B, 2), lambda i: (0, 0)),
        pl.BlockSpec((B, 1), lambda i: (0, 0)),
    )
    scores, ids, flag = pl.pallas_call(
        _subgraph_kernel,
        grid=(1,),
        in_specs=in_specs,
        out_specs=out_specs,
        out_shape=out_shapes,
    )(s_e, adjacency_matrix, W1, b1.reshape(1, D), W2, b2.reshape(1, 1))

    return ids, scores.reshape(B, 2, N), flag.reshape(B)
